# Initial kernel scaffold; baseline (speedup 1.0000x reference)
import jax
import jax.numpy as jnp
from jax.experimental import pallas as pl


def kernel(points, features, batch, global_features, mlp1_W1, mlp1_b1, mlp1_g1, mlp1_be1, mlp1_W2, mlp1_b2, mlp1_g2, mlp1_be2, mlp1_W3, mlp1_b3, mlp2_W1, mlp2_b1, mlp2_g1, mlp2_be1, mlp2_W2, mlp2_b2, head_W1, head_b1, head_W2, head_b2, head_W3, head_b3):
    raise NotImplementedError("write your pallas kernel here")


# speedup vs baseline: 4.8577x; 4.8577x over previous
import functools

import jax
import jax.numpy as jnp
from jax.experimental import pallas as pl
from jax.experimental.pallas import tpu as pltpu

B = 128
P = 128
K = 20
D = 3
FD = 16
EPS = 1e-5
E = float(B * P * K)
BIG = 1e30


def _dot(a, b):
    return jax.lax.dot_general(
        a, b, (((1,), (0,)), ((), ())),
        precision=jax.lax.Precision.HIGHEST,
        preferred_element_type=jnp.float32)


def _dott(a, b):
    return jax.lax.dot_general(
        a, b, (((1,), (1,)), ((), ())),
        precision=jax.lax.Precision.HIGHEST,
        preferred_element_type=jnp.float32)


def _relu(x):
    return jnp.maximum(x, 0.0)


def _knn_gather_loop(feat, v, u, body):
    c = feat.shape[1]
    ei = (jax.lax.broadcasted_iota(jnp.int32, (c, c), 0)
          == jax.lax.broadcasted_iota(jnp.int32, (c, c), 1)).astype(jnp.float32)
    feat_t = _dott(ei, feat)
    li = jax.lax.broadcasted_iota(jnp.int32, (P, P), 1)
    ri = jax.lax.broadcasted_iota(jnp.int32, (P, P), 0)
    d2 = jnp.zeros((P, P), jnp.float32)
    for d in range(c):
        diff = feat[:, d:d + 1] - feat_t[d:d + 1, :]
        d2 = d2 + diff * diff
    d2 = jnp.where(li == ri, BIG, d2)
    ki = jax.lax.broadcasted_iota(jnp.int32, (P, K), 1)
    idx_mat = jnp.zeros((P, K), jnp.int32)
    for k in range(K):
        m = jnp.min(d2, axis=1, keepdims=True)
        cand = jnp.where(d2 == m, li, jnp.int32(1 << 20))
        am = jnp.min(cand, axis=1, keepdims=True)
        oh = li == am
        d2 = jnp.where(oh, BIG, d2)
        idx_mat = jnp.where(ki == k, am, idx_mat)
        y = u + _dot(oh.astype(jnp.float32), v)
        body(k, y)
    return idx_mat


def _gather_loop(idx, v, u, body):
    li = jax.lax.broadcasted_iota(jnp.int32, (P, P), 1)
    for k in range(K):
        ic = idx[:, k:k + 1]
        oh = (li == ic).astype(jnp.float32)
        y = u + _dot(oh, v)
        body(k, y)


def _bn_affine(s_ref, q_ref, g_ref, be_ref):
    mean = s_ref[...] / E
    var = q_ref[...] / E - mean * mean
    inv = jax.lax.rsqrt(var + EPS)
    scale = g_ref[...] * inv
    shift = be_ref[...] - mean * scale
    return scale, shift


def _zero_on_first(ref):
    @pl.when(pl.program_id(0) == 0)
    def _():
        ref[...] = jnp.zeros_like(ref)


def _c1p1_kernel(pos_ref, x_ref, w1_ref, b1_ref,
                 idx_ref, u_ref, v_ref, s_ref, q_ref):
    pos = pos_ref[0]
    x = x_ref[0]
    w1 = w1_ref[...]
    wa = w1[0:FD, :]
    wb = w1[FD:2 * FD, :]
    u = _dot(x, wa - wb) + b1_ref[...]
    v = _dot(x, wb)
    u_ref[0] = u
    v_ref[0] = v
    _zero_on_first(s_ref)
    _zero_on_first(q_ref)
    acc = [jnp.zeros((1, 32), jnp.float32), jnp.zeros((1, 32), jnp.float32)]

    def body(k, y):
        acc[0] = acc[0] + jnp.sum(y, axis=0, keepdims=True)
        acc[1] = acc[1] + jnp.sum(y * y, axis=0, keepdims=True)

    idx_mat = _knn_gather_loop(pos, v, u, body)
    idx_ref[0] = idx_mat
    s_ref[...] = s_ref[...] + acc[0]
    q_ref[...] = q_ref[...] + acc[1]


def _c1p2_kernel(idx_ref, u_ref, v_ref, s1_ref, q1_ref, g1_ref, be1_ref,
                 w2_ref, b2_ref, s2_ref, q2_ref):
    scale1, shift1 = _bn_affine(s1_ref, q1_ref, g1_ref, be1_ref)
    u = u_ref[0]
    v = v_ref[0]
    w2 = w2_ref[...]
    b2 = b2_ref[...]
    _zero_on_first(s2_ref)
    _zero_on_first(q2_ref)
    acc = [jnp.zeros((1, 32), jnp.float32), jnp.zeros((1, 32), jnp.float32)]

    def body(k, y):
        h = _relu(y * scale1 + shift1)
        y2 = _dot(h, w2) + b2
        acc[0] = acc[0] + jnp.sum(y2, axis=0, keepdims=True)
        acc[1] = acc[1] + jnp.sum(y2 * y2, axis=0, keepdims=True)

    _gather_loop(idx_ref[0], v, u, body)
    s2_ref[...] = s2_ref[...] + acc[0]
    q2_ref[...] = q2_ref[...] + acc[1]


def _c1p3_kernel(idx_ref, u_ref, v_ref, s1_ref, q1_ref, g1_ref, be1_ref,
                 w2_ref, b2_ref, s2_ref, q2_ref, g2_ref, be2_ref,
                 w3_ref, b3_ref, x1_ref):
    scale1, shift1 = _bn_affine(s1_ref, q1_ref, g1_ref, be1_ref)
    scale2, shift2 = _bn_affine(s2_ref, q2_ref, g2_ref, be2_ref)
    u = u_ref[0]
    v = v_ref[0]
    w2 = w2_ref[...]
    b2 = b2_ref[...]
    w3 = w3_ref[...]
    b3 = b3_ref[...]
    acc = [jnp.zeros((P, 32), jnp.float32)]

    def body(k, y):
        h1 = _relu(y * scale1 + shift1)
        y2 = _dot(h1, w2) + b2
        h2 = _relu(y2 * scale2 + shift2)
        acc[0] = acc[0] + (_dot(h2, w3) + b3)

    _gather_loop(idx_ref[0], v, u, body)
    x1_ref[0] = acc[0] * (1.0 / K)


def _c2p1_kernel(x1_ref, w1_ref, b1_ref, idx_ref, u_ref, v_ref, s_ref, q_ref):
    x1 = x1_ref[0]
    w1 = w1_ref[...]
    wa = w1[0:32, :]
    wb = w1[32:64, :]
    u = _dot(x1, wa - wb) + b1_ref[...]
    v = _dot(x1, wb)
    u_ref[0] = u
    v_ref[0] = v
    _zero_on_first(s_ref)
    _zero_on_first(q_ref)
    acc = [jnp.zeros((1, 64), jnp.float32), jnp.zeros((1, 64), jnp.float32)]

    def body(k, y):
        acc[0] = acc[0] + jnp.sum(y, axis=0, keepdims=True)
        acc[1] = acc[1] + jnp.sum(y * y, axis=0, keepdims=True)

    idx_mat = _knn_gather_loop(x1, v, u, body)
    idx_ref[0] = idx_mat
    s_ref[...] = s_ref[...] + acc[0]
    q_ref[...] = q_ref[...] + acc[1]


def _c2p2_kernel(idx_ref, u_ref, v_ref, s_ref, q_ref, g_ref, be_ref,
                 w2_ref, b2_ref, pool_ref):
    scale, shift = _bn_affine(s_ref, q_ref, g_ref, be_ref)
    u = u_ref[0]
    v = v_ref[0]
    w2 = w2_ref[...]
    b2 = b2_ref[...]
    acc = [jnp.zeros((P, 64), jnp.float32)]

    def body(k, y):
        h = _relu(y * scale + shift)
        acc[0] = acc[0] + (_dot(h, w2) + b2)

    _gather_loop(idx_ref[0], v, u, body)
    x2 = acc[0] * (1.0 / K)
    pool_ref[0] = jnp.max(x2, axis=0, keepdims=True)


def _head_kernel(p_ref, w1_ref, b1_ref, w2_ref, b2_ref, w3_ref, b3_ref,
                 o_ref):
    h = _relu(_dot(p_ref[...], w1_ref[...]) + b1_ref[...])
    h = _relu(_dot(h, w2_ref[...]) + b2_ref[...])
    o_ref[...] = _dot(h, w3_ref[...]) + b3_ref[...]


def _full(shape):
    nd = len(shape)
    return pl.BlockSpec(shape, lambda g, _n=nd: (0,) * _n)


def _per_graph(shape):
    nd = len(shape)
    return pl.BlockSpec((1,) + shape, lambda g, _n=nd: (g,) + (0,) * _n)


def kernel(points, features, batch, global_features,
           mlp1_W1, mlp1_b1, mlp1_g1, mlp1_be1, mlp1_W2, mlp1_b2,
           mlp1_g2, mlp1_be2, mlp1_W3, mlp1_b3,
           mlp2_W1, mlp2_b1, mlp2_g1, mlp2_be1, mlp2_W2, mlp2_b2,
           head_W1, head_b1, head_W2, head_b2, head_W3, head_b3):
    f32 = jnp.float32
    pos = points.reshape(B, P, D)
    x = features.reshape(B, P, FD)
    row = lambda a: a.reshape(1, -1)

    grid = (B,)
    idx1, u1, v1, s1, q1 = pl.pallas_call(
        _c1p1_kernel,
        grid=grid,
        in_specs=[_per_graph((P, D)), _per_graph((P, FD)),
                  _full((2 * FD, 32)), _full((1, 32))],
        out_specs=[_per_graph((P, K)), _per_graph((P, 32)),
                   _per_graph((P, 32)), _full((1, 32)), _full((1, 32))],
        out_shape=[jax.ShapeDtypeStruct((B, P, K), jnp.int32),
                   jax.ShapeDtypeStruct((B, P, 32), f32),
                   jax.ShapeDtypeStruct((B, P, 32), f32),
                   jax.ShapeDtypeStruct((1, 32), f32),
                   jax.ShapeDtypeStruct((1, 32), f32)],
    )(pos, x, mlp1_W1, row(mlp1_b1))

    s2, q2 = pl.pallas_call(
        _c1p2_kernel,
        grid=grid,
        in_specs=[_per_graph((P, K)), _per_graph((P, 32)), _per_graph((P, 32)),
                  _full((1, 32)), _full((1, 32)), _full((1, 32)),
                  _full((1, 32)), _full((32, 32)), _full((1, 32))],
        out_specs=[_full((1, 32)), _full((1, 32))],
        out_shape=[jax.ShapeDtypeStruct((1, 32), f32),
                   jax.ShapeDtypeStruct((1, 32), f32)],
    )(idx1, u1, v1, s1, q1, row(mlp1_g1), row(mlp1_be1),
      mlp1_W2, row(mlp1_b2))

    x1 = pl.pallas_call(
        _c1p3_kernel,
        grid=grid,
        in_specs=[_per_graph((P, K)), _per_graph((P, 32)), _per_graph((P, 32)),
                  _full((1, 32)), _full((1, 32)), _full((1, 32)),
                  _full((1, 32)), _full((32, 32)), _full((1, 32)),
                  _full((1, 32)), _full((1, 32)), _full((1, 32)),
                  _full((1, 32)), _full((32, 32)), _full((1, 32))],
        out_specs=[_per_graph((P, 32))],
        out_shape=[jax.ShapeDtypeStruct((B, P, 32), f32)],
    )(idx1, u1, v1, s1, q1, row(mlp1_g1), row(mlp1_be1),
      mlp1_W2, row(mlp1_b2), s2, q2, row(mlp1_g2), row(mlp1_be2),
      mlp1_W3, row(mlp1_b3))[0]

    idx2, u2, v2, s3, q3 = pl.pallas_call(
        _c2p1_kernel,
        grid=grid,
        in_specs=[_per_graph((P, 32)), _full((64, 64)), _full((1, 64))],
        out_specs=[_per_graph((P, K)), _per_graph((P, 64)),
                   _per_graph((P, 64)), _full((1, 64)), _full((1, 64))],
        out_shape=[jax.ShapeDtypeStruct((B, P, K), jnp.int32),
                   jax.ShapeDtypeStruct((B, P, 64), f32),
                   jax.ShapeDtypeStruct((B, P, 64), f32),
                   jax.ShapeDtypeStruct((1, 64), f32),
                   jax.ShapeDtypeStruct((1, 64), f32)],
    )(x1, mlp2_W1, row(mlp2_b1))

    pooled = pl.pallas_call(
        _c2p2_kernel,
        grid=grid,
        in_specs=[_per_graph((P, K)), _per_graph((P, 64)), _per_graph((P, 64)),
                  _full((1, 64)), _full((1, 64)), _full((1, 64)),
                  _full((1, 64)), _full((64, 64)), _full((1, 64))],
        out_specs=[pl.BlockSpec((1, 1, 64), lambda g: (g, 0, 0))],
        out_shape=[jax.ShapeDtypeStruct((B, 1, 64), f32)],
    )(idx2, u2, v2, s3, q3, row(mlp2_g1), row(mlp2_be1),
      mlp2_W2, row(mlp2_b2))[0].reshape(B, 64)

    out = pl.pallas_call(
        _head_kernel,
        grid=(1,),
        in_specs=[_full((B, 64)), _full((64, 128)), _full((1, 128)),
                  _full((128, 128)), _full((1, 128)),
                  _full((128, 1)), _full((1, 1))],
        out_specs=[_full((B, 1))],
        out_shape=[jax.ShapeDtypeStruct((B, 1), f32)],
    )(pooled, head_W1, row(head_b1), head_W2, row(head_b2),
      head_W3, row(head_b3))[0]

    return out


# speedup vs baseline: 5.8085x; 1.1957x over previous
import jax
import jax.numpy as jnp
from jax.experimental import pallas as pl

B = 128
P = 128
K = 20
D = 3
FD = 16
EPS = 1e-5
E = float(B * P * K)
BIG = 1e30
G = 2
NSTEP = B // G


def _dot(a, b):
    return jax.lax.dot_general(
        a, b, (((1,), (0,)), ((), ())),
        precision=jax.lax.Precision.HIGHEST,
        preferred_element_type=jnp.float32)


def _dott(a, b):
    return jax.lax.dot_general(
        a, b, (((1,), (1,)), ((), ())),
        precision=jax.lax.Precision.HIGHEST,
        preferred_element_type=jnp.float32)


def _dot_bf(a, b):
    return jax.lax.dot_general(
        a.astype(jnp.bfloat16), b.astype(jnp.bfloat16),
        (((1,), (0,)), ((), ())),
        preferred_element_type=jnp.float32)


def _relu(x):
    return jnp.maximum(x, 0.0)


def _knn(feat):
    c = feat.shape[1]
    ei = (jax.lax.broadcasted_iota(jnp.int32, (c, c), 0)
          == jax.lax.broadcasted_iota(jnp.int32, (c, c), 1)).astype(jnp.float32)
    feat_t = _dott(ei, feat)
    li = jax.lax.broadcasted_iota(jnp.int32, (P, P), 1)
    ri = jax.lax.broadcasted_iota(jnp.int32, (P, P), 0)
    d2 = jnp.zeros((P, P), jnp.float32)
    for d in range(c):
        diff = feat[:, d:d + 1] - feat_t[d:d + 1, :]
        d2 = d2 + diff * diff
    d2 = jnp.where(li == ri, BIG, d2)
    ki = jax.lax.broadcasted_iota(jnp.int32, (P, K), 1)
    idx_mat = jnp.zeros((P, K), jnp.int32)
    for k in range(K):
        m = jnp.min(d2, axis=1, keepdims=True)
        cand = jnp.where(d2 == m, li, jnp.int32(1 << 20))
        am = jnp.min(cand, axis=1, keepdims=True)
        oh = li == am
        d2 = jnp.where(oh, BIG, d2)
        idx_mat = jnp.where(ki == k, am, idx_mat)
    return idx_mat


def _edge_msg(idx, x):
    li = jax.lax.broadcasted_iota(jnp.int32, (P, P), 1)
    oh = jnp.concatenate(
        [(li == idx[:, k:k + 1]).astype(jnp.float32) for k in range(K)],
        axis=0)
    xj = _dot(oh, x)
    xi = jnp.concatenate([x] * K, axis=0)
    return jnp.concatenate([xi, xj - xi], axis=1)


def _mean_over_k(y):
    c = y.shape[1]
    return jnp.sum(y.reshape(K, P, c), axis=0) * (1.0 / K)


def _bn_affine(st_ref, g_ref, be_ref):
    mean = st_ref[0:1, :] / E
    var = st_ref[2:3, :] / E - mean * mean
    inv = jax.lax.rsqrt(var + EPS)
    scale = g_ref[...] * inv
    shift = be_ref[...] - mean * scale
    return scale, shift


def _zero_on_first(ref):
    @pl.when(pl.program_id(0) == 0)
    def _():
        ref[...] = jnp.zeros_like(ref)


def _kahan_update(st_ref, s_contrib, q_contrib):
    for (r, x) in ((0, s_contrib), (2, q_contrib)):
        s = st_ref[r:r + 1, :]
        comp = st_ref[r + 1:r + 2, :]
        y = x - comp
        t = s + y
        st_ref[r + 1:r + 2, :] = (t - s) - y
        st_ref[r:r + 1, :] = t


class _Stats:
    def __init__(self, c):
        self.s = jnp.zeros((1, c), jnp.float32)
        self.q = jnp.zeros((1, c), jnp.float32)

    def add(self, y):
        self.s = self.s + jnp.sum(y, axis=0, keepdims=True)
        self.q = self.q + jnp.sum(y * y, axis=0, keepdims=True)


def _c1p1_kernel(pos_ref, x_ref, w1_ref, b1_ref, idx_ref, st_ref):
    _zero_on_first(st_ref)
    stats = _Stats(32)
    for g in range(G):
        idx_mat = _knn(pos_ref[g])
        idx_ref[g] = idx_mat
        y1 = _dot_bf(_edge_msg(idx_mat, x_ref[g]), w1_ref[...]) + b1_ref[...]
        stats.add(y1)
    _kahan_update(st_ref, stats.s, stats.q)


def _c1p2_kernel(x_ref, idx_ref, w1_ref, b1_ref, st1_ref, g1_ref, be1_ref,
                 w2_ref, b2_ref, st2_ref):
    scale1, shift1 = _bn_affine(st1_ref, g1_ref, be1_ref)
    _zero_on_first(st2_ref)
    stats = _Stats(32)
    for g in range(G):
        y1 = _dot_bf(_edge_msg(idx_ref[g], x_ref[g]), w1_ref[...]) + b1_ref[...]
        h1 = _relu(y1 * scale1 + shift1)
        stats.add(_dot_bf(h1, w2_ref[...]) + b2_ref[...])
    _kahan_update(st2_ref, stats.s, stats.q)


def _c1p3_c2p1_kernel(x_ref, idx_ref, w1_ref, b1_ref, st1_ref, g1_ref,
                      be1_ref, w2_ref, b2_ref, st2_ref, g2_ref, be2_ref,
                      w3_ref, b3_ref, m2w1_ref, m2b1_ref,
                      idx2_ref, x1_ref, st3_ref):
    scale1, shift1 = _bn_affine(st1_ref, g1_ref, be1_ref)
    scale2, shift2 = _bn_affine(st2_ref, g2_ref, be2_ref)
    _zero_on_first(st3_ref)
    stats = _Stats(64)
    for g in range(G):
        y1 = _dot_bf(_edge_msg(idx_ref[g], x_ref[g]), w1_ref[...]) + b1_ref[...]
        h1 = _relu(y1 * scale1 + shift1)
        y2 = _dot_bf(h1, w2_ref[...]) + b2_ref[...]
        h2 = _relu(y2 * scale2 + shift2)
        y3 = _dot_bf(h2, w3_ref[...]) + b3_ref[...]
        x1 = _mean_over_k(y3)
        x1_ref[g] = x1
        idx2_mat = _knn(x1)
        idx2_ref[g] = idx2_mat
        ym = _dot_bf(_edge_msg(idx2_mat, x1), m2w1_ref[...]) + m2b1_ref[...]
        stats.add(ym)
    _kahan_update(st3_ref, stats.s, stats.q)


def _c2p2_kernel(x1_ref, idx2_ref, w1_ref, b1_ref, st_ref, g_ref, be_ref,
                 w2_ref, b2_ref, pool_ref):
    scale, shift = _bn_affine(st_ref, g_ref, be_ref)
    for g in range(G):
        y1 = _dot_bf(_edge_msg(idx2_ref[g], x1_ref[g]),
                     w1_ref[...]) + b1_ref[...]
        h = _relu(y1 * scale + shift)
        y2 = _dot_bf(h, w2_ref[...]) + b2_ref[...]
        x2 = _mean_over_k(y2)
        pool_ref[g] = jnp.max(x2, axis=0, keepdims=True)


def _head_kernel(p_ref, w1_ref, b1_ref, w2_ref, b2_ref, w3_ref, b3_ref,
                 o_ref):
    h = _relu(_dot_bf(p_ref[...], w1_ref[...]) + b1_ref[...])
    h = _relu(_dot_bf(h, w2_ref[...]) + b2_ref[...])
    o_ref[...] = _dot_bf(h, w3_ref[...]) + b3_ref[...]


def _full(shape):
    nd = len(shape)
    return pl.BlockSpec(shape, lambda g, _n=nd: (0,) * _n)


def _per_graph(shape):
    nd = len(shape)
    return pl.BlockSpec((G,) + shape, lambda g, _n=nd: (g,) + (0,) * _n)


def kernel(points, features, batch, global_features,
           mlp1_W1, mlp1_b1, mlp1_g1, mlp1_be1, mlp1_W2, mlp1_b2,
           mlp1_g2, mlp1_be2, mlp1_W3, mlp1_b3,
           mlp2_W1, mlp2_b1, mlp2_g1, mlp2_be1, mlp2_W2, mlp2_b2,
           head_W1, head_b1, head_W2, head_b2, head_W3, head_b3):
    f32 = jnp.float32
    pos = points.reshape(B, P, D)
    x = features.reshape(B, P, FD)
    row = lambda a: a.reshape(1, -1)

    grid = (NSTEP,)
    idx1, st1 = pl.pallas_call(
        _c1p1_kernel,
        grid=grid,
        in_specs=[_per_graph((P, D)), _per_graph((P, FD)),
                  _full((2 * FD, 32)), _full((1, 32))],
        out_specs=[_per_graph((P, K)), _full((4, 32))],
        out_shape=[jax.ShapeDtypeStruct((B, P, K), jnp.int32),
                   jax.ShapeDtypeStruct((4, 32), f32)],
    )(pos, x, mlp1_W1, row(mlp1_b1))

    st2 = pl.pallas_call(
        _c1p2_kernel,
        grid=grid,
        in_specs=[_per_graph((P, FD)), _per_graph((P, K)),
                  _full((2 * FD, 32)), _full((1, 32)), _full((4, 32)),
                  _full((1, 32)), _full((1, 32)),
                  _full((32, 32)), _full((1, 32))],
        out_specs=[_full((4, 32))],
        out_shape=[jax.ShapeDtypeStruct((4, 32), f32)],
    )(x, idx1, mlp1_W1, row(mlp1_b1), st1, row(mlp1_g1), row(mlp1_be1),
      mlp1_W2, row(mlp1_b2))[0]

    idx2, x1, st3 = pl.pallas_call(
        _c1p3_c2p1_kernel,
        grid=grid,
        in_specs=[_per_graph((P, FD)), _per_graph((P, K)),
                  _full((2 * FD, 32)), _full((1, 32)), _full((4, 32)),
                  _full((1, 32)), _full((1, 32)),
                  _full((32, 32)), _full((1, 32)),
                  _full((4, 32)), _full((1, 32)), _full((1, 32)),
                  _full((32, 32)), _full((1, 32)),
                  _full((64, 64)), _full((1, 64))],
        out_specs=[_per_graph((P, K)), _per_graph((P, 32)), _full((4, 64))],
        out_shape=[jax.ShapeDtypeStruct((B, P, K), jnp.int32),
                   jax.ShapeDtypeStruct((B, P, 32), f32),
                   jax.ShapeDtypeStruct((4, 64), f32)],
    )(x, idx1, mlp1_W1, row(mlp1_b1), st1, row(mlp1_g1), row(mlp1_be1),
      mlp1_W2, row(mlp1_b2), st2, row(mlp1_g2), row(mlp1_be2),
      mlp1_W3, row(mlp1_b3), mlp2_W1, row(mlp2_b1))

    pooled = pl.pallas_call(
        _c2p2_kernel,
        grid=grid,
        in_specs=[_per_graph((P, 32)), _per_graph((P, K)),
                  _full((64, 64)), _full((1, 64)), _full((4, 64)),
                  _full((1, 64)), _full((1, 64)),
                  _full((64, 64)), _full((1, 64))],
        out_specs=[pl.BlockSpec((G, 1, 64), lambda g: (g, 0, 0))],
        out_shape=[jax.ShapeDtypeStruct((B, 1, 64), f32)],
    )(x1, idx2, mlp2_W1, row(mlp2_b1), st3, row(mlp2_g1), row(mlp2_be1),
      mlp2_W2, row(mlp2_b2))[0].reshape(B, 64)

    out = pl.pallas_call(
        _head_kernel,
        grid=(1,),
        in_specs=[_full((B, 64)), _full((64, 128)), _full((1, 128)),
                  _full((128, 128)), _full((1, 128)),
                  _full((128, 1)), _full((1, 1))],
        out_specs=[_full((B, 1))],
        out_shape=[jax.ShapeDtypeStruct((B, 1), f32)],
    )(pooled, head_W1, row(head_b1), head_W2, row(head_b2),
      head_W3, row(head_b3))[0]

    return out
